# baseline (speedup 1.0000x reference)
import functools

import jax
import jax.numpy as jnp
from jax.experimental import pallas as pl
from jax.experimental.pallas import tpu as pltpu

B, L, V, D = 1024, 50, 100000, 64
K = 100
BT = 512
VT = 2048
GRID_V = (V + VT - 1) // VT


def _logits_kernel(pooled_ref, w_ref, tgt_ref, out_ref, tl_ref):
    j = pl.program_id(1)
    logits = jax.lax.dot_general(
        pooled_ref[...], w_ref[...],
        (((1,), (1,)), ((), ())),
        preferred_element_type=jnp.float32,
    )
    cols = j * VT + jax.lax.broadcasted_iota(jnp.int32, (BT, VT), 1)
    match = cols == tgt_ref[...]
    out_ref[...] = jnp.where(match, 0.0, logits)
    contrib = jnp.sum(jnp.where(match, logits, 0.0), axis=1, keepdims=True)

    @pl.when(j == 0)
    def _():
        tl_ref[...] = contrib

    @pl.when(j > 0)
    def _():
        tl_ref[...] += contrib


def _masked_logits(pooled_q, Wq, target_id):
    tgt2d = target_id.reshape(B, 1).astype(jnp.int32)
    return pl.pallas_call(
        _logits_kernel,
        grid=(B // BT, GRID_V),
        in_specs=[
            pl.BlockSpec((BT, D), lambda i, j: (i, 0)),
            pl.BlockSpec((VT, D), lambda i, j: (j, 0)),
            pl.BlockSpec((BT, 1), lambda i, j: (i, 0)),
        ],
        out_specs=[
            pl.BlockSpec((BT, VT), lambda i, j: (i, j)),
            pl.BlockSpec((BT, 1), lambda i, j: (i, 0)),
        ],
        out_shape=[
            jax.ShapeDtypeStruct((B, V), jnp.float32),
            jax.ShapeDtypeStruct((B, 1), jnp.float32),
        ],
    )(pooled_q, Wq, tgt2d)


def _loss_kernel(noise_ref, actual_ref, out_ref):
    noise = noise_ref[...]
    actual = actual_ref[...]

    def softmax(x):
        m = jnp.max(x, axis=1, keepdims=True)
        e = jnp.exp(x - m)
        return e / jnp.sum(e, axis=1, keepdims=True)

    n_sm = softmax(noise)
    a_sm = softmax(actual)
    deno = K * n_sm + a_sm
    tmp1 = a_sm / deno
    tmp2 = n_sm / deno
    pos = jax.lax.broadcasted_iota(jnp.int32, (B, K + 1), 1)
    likeli = jnp.where(pos == 0, tmp1, tmp2)
    loss = -jnp.sum(jnp.log(likeli)) / (B * (K + 1))
    out_ref[...] = loss.reshape(1, 1)


def _nce_loss(noise_raw, actual_raw):
    out = pl.pallas_call(
        _loss_kernel,
        out_shape=jax.ShapeDtypeStruct((1, 1), jnp.float32),
    )(noise_raw, actual_raw)
    return out[0, 0]


def _pool(E, item_seq, item_seq_len):
    h = jnp.take(E, item_seq, axis=0)
    mask = (jnp.arange(L)[None, :] < item_seq_len[:, None]).astype(h.dtype)
    denom = jnp.maximum(item_seq_len, 1).astype(h.dtype)[:, None]
    return (h * mask[:, :, None]).sum(axis=1) / denom


def kernel(Ep, Wp, Eq, Wq, item_seq, item_seq_len, target_id):
    pooled_q = _pool(Eq, item_seq, item_seq_len)
    pooled_p = _pool(Ep, item_seq, item_seq_len)

    masked_logits, target_logit = _masked_logits(pooled_q, Wq, target_id)

    _, topk_idx = jax.lax.top_k(masked_logits, K)
    indices = jnp.concatenate([target_id[:, None], topk_idx], axis=1)

    noise_raw = jnp.take_along_axis(masked_logits, indices, axis=1)
    noise_raw = jnp.where(indices == target_id[:, None], target_logit, noise_raw)

    wp_rows = jnp.take(Wp, indices, axis=0)
    actual_raw = jnp.einsum("bd,bjd->bj", pooled_p, wp_rows)

    return _nce_loss(noise_raw, actual_raw)


# speedup vs baseline: 8.2962x; 8.2962x over previous
import functools

import numpy as np

import jax
import jax.numpy as jnp
from jax import lax
from jax.experimental import pallas as pl
from jax.experimental.pallas import tpu as pltpu
from jax.experimental.pallas import tpu_sc as plsc

B, L, V, D = 1024, 50, 100000, 64
K = 100
BT = 512
VT = 2048
GRID_V = (V + VT - 1) // VT


def _logits_kernel(pooled_ref, w_ref, tgt_ref, out_ref, tl_ref):
    j = pl.program_id(1)
    logits = jax.lax.dot_general(
        pooled_ref[...], w_ref[...],
        (((1,), (1,)), ((), ())),
        preferred_element_type=jnp.float32,
    )
    cols = j * VT + jax.lax.broadcasted_iota(jnp.int32, (BT, VT), 1)
    match = cols == tgt_ref[...]
    masked = jnp.where(match, 0.0, logits)
    masked = jnp.where(cols < V, masked, NEG)
    out_ref[...] = masked.reshape(BT, VT // 128, 128)
    contrib = jnp.sum(jnp.where(jnp.logical_and(match, cols < V), logits, 0.0),
                      axis=1, keepdims=True)

    @pl.when(j == 0)
    def _():
        tl_ref[...] = contrib

    @pl.when(j > 0)
    def _():
        tl_ref[...] += contrib


def _masked_logits(pooled_q, Wq, target_id):
    tgt2d = target_id.reshape(B, 1).astype(jnp.int32)
    return pl.pallas_call(
        _logits_kernel,
        grid=(B // BT, GRID_V),
        in_specs=[
            pl.BlockSpec((BT, D), lambda i, j: (i, 0)),
            pl.BlockSpec((VT, D), lambda i, j: (j, 0)),
            pl.BlockSpec((BT, 1), lambda i, j: (i, 0)),
        ],
        out_specs=[
            pl.BlockSpec((BT, VT // 128, 128), lambda i, j: (i, j, 0)),
            pl.BlockSpec((BT, 1), lambda i, j: (i, 0)),
        ],
        out_shape=[
            jax.ShapeDtypeStruct((B, VPAD // 128, 128), jnp.float32),
            jax.ShapeDtypeStruct((B, 1), jnp.float32),
        ],
    )(pooled_q, Wq, tgt2d)


NC, NS, LN = 2, 16, 16
NW = NC * NS
RPW = B // NW
NBLK = 784
VPAD = NBLK * 128
CAP = 4096
OUTW = 104
NEG = -3.0e38
TOPBIT = np.uint32(0x80000000)


_GATHER_DNUMS = lax.GatherDimensionNumbers(
    offset_dims=(), collapsed_slice_dims=(0,), start_index_map=(0,))


def _perm16(x, perm):
    return lax.gather(x, perm.reshape(16, 1), _GATHER_DNUMS, (1,),
                      mode=lax.GatherScatterMode.PROMISE_IN_BOUNDS)


def _allmax16(x, lane):
    for sh in (8, 4, 2, 1):
        x = jnp.maximum(x, _perm16(x, (lane + sh) & 15))
    return x


def _allmin16(x, lane):
    for sh in (8, 4, 2, 1):
        x = jnp.minimum(x, _perm16(x, (lane + sh) & 15))
    return x


def _allsum16(x, lane):
    for sh in (8, 4, 2, 1):
        x = x + _perm16(x, (lane + sh) & 15)
    return x


def _cumsum16(x, lane):
    for sh in (1, 2, 4, 8):
        shifted = _perm16(x, (lane - sh) & 15)
        x = x + jnp.where(lane >= sh, shifted, 0)
    return x


def _compact16(arrs, msk, lane):
    rank = _cumsum16(jnp.where(msk, 1, 0), lane)
    cnt = rank[15]
    d = jnp.where(msk, lane - (rank - 1), 0)
    for k in (1, 2, 4, 8):
        perm = (lane + k) & 15
        d_s = _perm16(d, perm)
        take = jnp.logical_and(lane < 16 - k, (d_s & k) != 0)
        arrs = [jnp.where(take, _perm16(a, perm), a) for a in arrs]
        d = jnp.where(take, d_s - k, jnp.where((d & k) != 0, 0, d))
    return arrs, cnt


def _skey16(x):
    u = lax.bitcast_convert_type(x, jnp.int32)
    return u ^ (lax.shift_right_arithmetic(u, 31) & jnp.int32(0x7FFFFFFF))


def _sval16(k):
    u = k ^ (lax.shift_right_arithmetic(k, 31) & jnp.int32(0x7FFFFFFF))
    return lax.bitcast_convert_type(u, jnp.float32)


def _topk_body(logits_hbm, tgt_hbm, tl_hbm, idx_out, val_out,
               row_v, bm_v, qblk_v, cmono_v, cidx_v, sidx_v, skey_v, sval_v,
               tgt_v, tl_v, sem):
    wid = lax.axis_index("s") * NC + lax.axis_index("c")
    lane = lax.iota(jnp.int32, 16)

    pltpu.sync_copy(tgt_hbm.at[pl.ds(wid * RPW, RPW)], tgt_v)
    pltpu.sync_copy(tl_hbm.at[pl.ds(wid * RPW, RPW)], tl_v)

    def row_body(r, _):
        b = wid * RPW + r
        pltpu.sync_copy(logits_hbm.at[b], row_v)

        def bm_group(g, carry):
            gminv, gmaxv = carry
            acc = jnp.full((LN,), NEG, jnp.float32)
            for j in range(16):
                m = row_v[g * 16 + j, pl.ds(0, 16)]
                for t in range(1, 8):
                    m = jnp.maximum(m, row_v[g * 16 + j, pl.ds(t * 16, 16)])
                s = _allmax16(m, lane)[0]
                acc = jnp.where(lane == j, s, acc)
            bm_v[pl.ds(g * 16, 16)] = acc
            valid = (g * 16 + lane) < (V + 127) // 128
            gminv = jnp.minimum(gminv, jnp.where(valid, acc, -NEG))
            gmaxv = jnp.maximum(gmaxv, jnp.where(valid, acc, NEG))
            return gminv, gmaxv

        gminv, gmaxv = lax.fori_loop(
            0, NBLK // 16, bm_group,
            (jnp.full((LN,), -NEG, jnp.float32), jnp.full((LN,), NEG, jnp.float32)))
        gmin = _allmin16(gminv, lane)[0]
        gmax = _allmax16(gmaxv, lane)[0]

        def count_bm(t):
            def cb(v, c):
                return c + jnp.where(bm_v[pl.ds(v * 16, 16)] >= t, 1, 0)
            return _allsum16(
                lax.fori_loop(0, NBLK // 16, cb, jnp.zeros((LN,), jnp.int32)), lane)[0]

        def s1_body(_, st):
            lo, hi, best, done = st
            mid = 0.5 * (lo + hi)
            cnt = count_bm(mid)
            feas = jnp.logical_and(cnt >= K, done == 0)
            best = jnp.where(feas, mid, best)
            lo2 = jnp.where(feas, mid, lo)
            hi2 = jnp.where(jnp.logical_or(feas, done != 0), hi, mid)
            done2 = jnp.where(jnp.logical_and(feas, cnt <= 160),
                              jnp.int32(1), done)
            return lo2, hi2, best, done2

        _, _, g_thr, _ = lax.fori_loop(
            0, 18, s1_body, (gmin, gmax, gmin, jnp.int32(0)))

        def append16(ref, pend, vals, off, cnt, trash, cap):
            sh = off & 15
            rot = [_perm16(v, (lane - sh) & 15) for v in vals]
            merged = [jnp.where(lane < sh, p, r) for p, r in zip(pend, rot)]
            full = (sh + cnt) >= 16
            dst = jnp.where(full, jnp.minimum(off - sh, cap), trash)
            for rf, m in zip(ref, merged):
                rf[pl.ds(dst, 16)] = m
            pend2 = [jnp.where(full, r, m) for r, m in zip(rot, merged)]
            return pend2, off + cnt

        def flush16(ref, pend, off, cap):
            sh = off & 15
            dst = jnp.minimum(off - sh, cap)
            for rf, p in zip(ref, pend):
                rf[pl.ds(dst, 16)] = p

        def qual_grp(grp, carry):
            qoff, pend = carry
            bmv = bm_v[pl.ds(grp * 16, 16)]
            msk = bmv >= g_thr
            (blks,), cnt = _compact16([grp * 16 + lane], msk, lane)
            (pend2,), qoff2 = append16([qblk_v], [pend], [blks], qoff, cnt,
                                       NBLK + 16, NBLK - 16)
            return qoff2, pend2

        nq, qpend = lax.fori_loop(0, NBLK // 16, qual_grp,
                                  (jnp.int32(0), lane))
        flush16([qblk_v], [qpend], nq, NBLK - 16)

        def coll_blk(q, vc):
            qv = qblk_v[pl.ds(q - (q & 15), 16)]
            blk = _allsum16(jnp.where(lane == (q & 15), qv, 0), lane)[0]
            for t in range(8):
                x = row_v[blk, pl.ds(t * 16, 16)]
                msk = x >= g_thr
                anyhit = _allsum16(jnp.where(msk, 1, 0), lane)[0]

                def do_store(vc, x=x, msk=msk, blk=blk, t=t):
                    vcw = jnp.minimum(vc, CAP // 16 - 1)
                    cmono_v[pl.ds(vcw * 16, 16)] = jnp.where(
                        msk, _skey16(x), jnp.int32(-2147483648))
                    cidx_v[pl.ds(vcw * 16, 16)] = blk * 128 + t * 16 + lane
                    return vc + 1

                vc = lax.cond(anyhit > 0, do_store, lambda v: v, vc)
            return vc

        nv = plsc.parallel_loop(0, nq, carry=jnp.int32(0))(coll_blk)
        n = nv * 16

        def count_cand(t, strict):
            def cb(v, c):
                key = cmono_v[pl.ds(v * 16, 16)]
                valid = (v * 16 + lane) < n
                hit = jnp.where(strict, key > t, key >= t)
                return c + jnp.where(jnp.logical_and(valid, hit), 1, 0)
            cvec = plsc.parallel_loop(0, nv, carry=jnp.zeros((LN,), jnp.int32))(cb)
            return _allsum16(cvec, lane)[0]

        def s2_body(_, st):
            lo, hi = st
            mid = (lo & hi) + ((lo ^ hi) >> 1) + ((lo ^ hi) & 1)
            feas = jnp.logical_and(lo < hi, count_cand(mid, jnp.bool_(False)) >= K)
            return (jnp.where(feas, mid, lo),
                    jnp.where(jnp.logical_or(feas, lo >= hi), hi, mid - 1))

        tau, _ = lax.fori_loop(
            0, 32, s2_body,
            (jnp.int32(-2147483648), jnp.int32(2147483647)))
        c_gt = count_cand(tau, jnp.bool_(True))
        rem = K - c_gt

        def sel_body(v, carry):
            off2, eqseen, pk, pi = carry
            key = cmono_v[pl.ds(v * 16, 16)]
            valid = (v * 16 + lane) < n
            gt = jnp.logical_and(valid, key > tau)
            eq = jnp.logical_and(valid, key == tau)
            eqr = _cumsum16(jnp.where(eq, 1, 0), lane) + eqseen
            sel = jnp.logical_or(gt, jnp.logical_and(eq, eqr <= rem))
            idx16 = cidx_v[pl.ds(v * 16, 16)]
            (key_s, idx_s), cnt = _compact16([key, idx16], sel, lane)
            (pk, pi), off2b = append16([skey_v, sidx_v], [pk, pi],
                                       [key_s, idx_s], off2, cnt, 144, 96)
            return off2b, eqr[15], pk, pi

        off2, _, spk, spi = plsc.parallel_loop(
            0, nv, carry=(jnp.int32(1), jnp.int32(0), lane, lane))(sel_body)
        flush16([skey_v, sidx_v], [spk, spi], off2, 96)

        rlane = r & 15
        rbase = r - rlane
        tgt = _allsum16(jnp.where(lane == rlane, tgt_v[pl.ds(rbase, 16)], 0), lane)[0]
        tl = _allsum16(jnp.where(lane == rlane, tl_v[pl.ds(rbase, 16)], 0.0), lane)[0]
        first_idx = jnp.where(lane == 0, tgt, sidx_v[pl.ds(0, 16)])
        sidx_v[pl.ds(0, 16)] = first_idx
        for i in range(OUTW // 8 // 2):
            ix = sidx_v[pl.ds(i * 16, 16)]
            vv = _sval16(skey_v[pl.ds(i * 16, 16)])
            sval_v[pl.ds(i * 16, 16)] = jnp.where(ix == tgt, tl, vv)

        pltpu.sync_copy(sidx_v.at[pl.ds(0, 128)], idx_out.at[pl.ds(b * 128, 128)])
        pltpu.sync_copy(sval_v.at[pl.ds(0, 128)], val_out.at[pl.ds(b * 128, 128)])
        return 0

    lax.fori_loop(0, RPW, row_body, 0)


def _sc_topk(masked_logits, target_id, target_logit):
    fn = pl.kernel(
        _topk_body,
        out_type=[jax.ShapeDtypeStruct((B * 128,), jnp.int32),
                  jax.ShapeDtypeStruct((B * 128,), jnp.float32)],
        mesh=plsc.VectorSubcoreMesh(core_axis_name="c", subcore_axis_name="s"),
        scratch_types=[
            pltpu.VMEM((NBLK, 128), jnp.float32),
            pltpu.VMEM((NBLK,), jnp.float32),
            pltpu.VMEM((NBLK + 48,), jnp.int32),
            pltpu.VMEM((CAP + 48,), jnp.int32),
            pltpu.VMEM((CAP + 48,), jnp.int32),
            pltpu.VMEM((256,), jnp.int32),
            pltpu.VMEM((256,), jnp.int32),
            pltpu.VMEM((256,), jnp.float32),
            pltpu.VMEM((RPW,), jnp.int32),
            pltpu.VMEM((RPW,), jnp.float32),
            pltpu.SemaphoreType.DMA,
        ],
    )
    return fn(masked_logits, target_id.astype(jnp.int32),
              target_logit.reshape(B))


def _loss_kernel(noise_ref, actual_ref, out_ref):
    noise = noise_ref[...]
    actual = actual_ref[...]

    def softmax(x):
        m = jnp.max(x, axis=1, keepdims=True)
        e = jnp.exp(x - m)
        return e / jnp.sum(e, axis=1, keepdims=True)

    n_sm = softmax(noise)
    a_sm = softmax(actual)
    deno = K * n_sm + a_sm
    tmp1 = a_sm / deno
    tmp2 = n_sm / deno
    pos = jax.lax.broadcasted_iota(jnp.int32, (B, K + 1), 1)
    likeli = jnp.where(pos == 0, tmp1, tmp2)
    loss = -jnp.sum(jnp.log(likeli)) / (B * (K + 1))
    out_ref[...] = loss.reshape(1, 1)


def _nce_loss(noise_raw, actual_raw):
    out = pl.pallas_call(
        _loss_kernel,
        out_shape=jax.ShapeDtypeStruct((1, 1), jnp.float32),
    )(noise_raw, actual_raw)
    return out[0, 0]


def _pool(E, item_seq, item_seq_len):
    h = jnp.take(E, item_seq, axis=0)
    mask = (jnp.arange(L)[None, :] < item_seq_len[:, None]).astype(h.dtype)
    denom = jnp.maximum(item_seq_len, 1).astype(h.dtype)[:, None]
    return (h * mask[:, :, None]).sum(axis=1) / denom


def kernel(Ep, Wp, Eq, Wq, item_seq, item_seq_len, target_id):
    pooled_q = _pool(Eq, item_seq, item_seq_len)
    pooled_p = _pool(Ep, item_seq, item_seq_len)

    masked_logits, target_logit = _masked_logits(pooled_q, Wq, target_id)

    idx_out, val_out = _sc_topk(masked_logits, target_id, target_logit)
    indices = idx_out.reshape(B, 128)[:, :K + 1]
    noise_raw = val_out.reshape(B, 128)[:, :K + 1]

    wp_rows = jnp.take(Wp, indices, axis=0)
    actual_raw = jnp.einsum("bd,bjd->bj", pooled_p, wp_rows)

    return _nce_loss(noise_raw, actual_raw)


# speedup vs baseline: 9.6014x; 1.1573x over previous
import functools

import numpy as np

import jax
import jax.numpy as jnp
from jax import lax
from jax.experimental import pallas as pl
from jax.experimental.pallas import tpu as pltpu
from jax.experimental.pallas import tpu_sc as plsc

B, L, V, D = 1024, 50, 100000, 64
K = 100
BT = 512
VT = 2048
GRID_V = (V + VT - 1) // VT


def _logits_kernel(pooled_ref, w_ref, tgt_ref, out_ref, tl_ref):
    j = pl.program_id(1)
    logits = jax.lax.dot_general(
        pooled_ref[...], w_ref[...],
        (((1,), (1,)), ((), ())),
        preferred_element_type=jnp.float32,
    )
    cols = j * VT + jax.lax.broadcasted_iota(jnp.int32, (BT, VT), 1)
    match = cols == tgt_ref[...]
    masked = jnp.where(match, 0.0, logits)
    masked = jnp.where(cols < V, masked, NEG)
    out_ref[...] = masked.reshape(BT, VT // 128, 128)
    contrib = jnp.sum(jnp.where(jnp.logical_and(match, cols < V), logits, 0.0),
                      axis=1, keepdims=True)

    @pl.when(j == 0)
    def _():
        tl_ref[...] = contrib

    @pl.when(j > 0)
    def _():
        tl_ref[...] += contrib


def _masked_logits(pooled_q, Wq, target_id):
    tgt2d = target_id.reshape(B, 1).astype(jnp.int32)
    return pl.pallas_call(
        _logits_kernel,
        grid=(B // BT, GRID_V),
        in_specs=[
            pl.BlockSpec((BT, D), lambda i, j: (i, 0)),
            pl.BlockSpec((VT, D), lambda i, j: (j, 0)),
            pl.BlockSpec((BT, 1), lambda i, j: (i, 0)),
        ],
        out_specs=[
            pl.BlockSpec((BT, VT // 128, 128), lambda i, j: (i, j, 0)),
            pl.BlockSpec((BT, 1), lambda i, j: (i, 0)),
        ],
        out_shape=[
            jax.ShapeDtypeStruct((B, VPAD // 128, 128), jnp.float32),
            jax.ShapeDtypeStruct((B, 1), jnp.float32),
        ],
    )(pooled_q, Wq, tgt2d)


NC, NS, LN = 2, 16, 16
NW = NC * NS
RPW = B // NW
NBLK = 784
VPAD = NBLK * 128
CAP = 2048
OUTW = 104
NEG = -3.0e38
TOPBIT = np.uint32(0x80000000)


_GATHER_DNUMS = lax.GatherDimensionNumbers(
    offset_dims=(), collapsed_slice_dims=(0,), start_index_map=(0,))


def _perm16(x, perm):
    return lax.gather(x, perm.reshape(16, 1), _GATHER_DNUMS, (1,),
                      mode=lax.GatherScatterMode.PROMISE_IN_BOUNDS)


def _allmax16(x, lane):
    for sh in (8, 4, 2, 1):
        x = jnp.maximum(x, _perm16(x, (lane + sh) & 15))
    return x


def _allmin16(x, lane):
    for sh in (8, 4, 2, 1):
        x = jnp.minimum(x, _perm16(x, (lane + sh) & 15))
    return x


def _allsum16(x, lane):
    for sh in (8, 4, 2, 1):
        x = x + _perm16(x, (lane + sh) & 15)
    return x


def _cumsum16(x, lane):
    for sh in (1, 2, 4, 8):
        shifted = _perm16(x, (lane - sh) & 15)
        x = x + jnp.where(lane >= sh, shifted, 0)
    return x


def _compact16(arrs, msk, lane):
    rank = _cumsum16(jnp.where(msk, 1, 0), lane)
    cnt = rank[15]
    d = jnp.where(msk, lane - (rank - 1), 0)
    for k in (1, 2, 4, 8):
        perm = (lane + k) & 15
        d_s = _perm16(d, perm)
        take = jnp.logical_and(lane < 16 - k, (d_s & k) != 0)
        arrs = [jnp.where(take, _perm16(a, perm), a) for a in arrs]
        d = jnp.where(take, d_s - k, jnp.where((d & k) != 0, 0, d))
    return arrs, cnt


def _skey16(x):
    u = lax.bitcast_convert_type(x, jnp.int32)
    return u ^ (lax.shift_right_arithmetic(u, 31) & jnp.int32(0x7FFFFFFF))


def _sval16(k):
    u = k ^ (lax.shift_right_arithmetic(k, 31) & jnp.int32(0x7FFFFFFF))
    return lax.bitcast_convert_type(u, jnp.float32)


def _topk_body(logits_hbm, tgt_hbm, tl_hbm, idx_out, val_out,
               row_v, bm_v, qblk_v, cmono_v, cidx_v, sidx_v, skey_v, sval_v,
               tgt_v, tl_v, sem, sem2):
    wid = lax.axis_index("s") * NC + lax.axis_index("c")
    lane = lax.iota(jnp.int32, 16)

    pltpu.sync_copy(tgt_hbm.at[pl.ds(wid * RPW, RPW)], tgt_v)
    pltpu.sync_copy(tl_hbm.at[pl.ds(wid * RPW, RPW)], tl_v)

    def row_body(r, _):
        b = wid * RPW + r
        cp0 = pltpu.async_copy(logits_hbm.at[b, pl.ds(0, 384)],
                               row_v.at[pl.ds(0, 384)], sem)
        cp1 = pltpu.async_copy(logits_hbm.at[b, pl.ds(384, 400)],
                               row_v.at[pl.ds(384, 400)], sem2)

        def bm_group(g, carry):
            gminv, gmaxv = carry
            acc = jnp.full((LN,), NEG, jnp.float32)
            for j in range(16):
                m = row_v[g * 16 + j, pl.ds(0, 16)]
                for t in range(1, 8):
                    m = jnp.maximum(m, row_v[g * 16 + j, pl.ds(t * 16, 16)])
                s = _allmax16(m, lane)[0]
                acc = jnp.where(lane == j, s, acc)
            bm_v[pl.ds(g * 16, 16)] = acc
            valid = (g * 16 + lane) < (V + 127) // 128
            gminv = jnp.minimum(gminv, jnp.where(valid, acc, -NEG))
            gmaxv = jnp.maximum(gmaxv, jnp.where(valid, acc, NEG))
            return gminv, gmaxv

        cp0.wait()
        init = (jnp.full((LN,), -NEG, jnp.float32),
                jnp.full((LN,), NEG, jnp.float32))
        carry0 = lax.fori_loop(0, 24, bm_group, init)
        cp1.wait()
        gminv, gmaxv = lax.fori_loop(24, NBLK // 16, bm_group, carry0)
        gmin = _allmin16(gminv, lane)[0]
        gmax = _allmax16(gmaxv, lane)[0]

        def count_bm(t):
            def cb(v, c):
                return c + jnp.where(bm_v[pl.ds(v * 16, 16)] >= t, 1, 0)
            return _allsum16(
                lax.fori_loop(0, NBLK // 16, cb, jnp.zeros((LN,), jnp.int32)), lane)[0]

        def s1_body(_, st):
            lo, hi, best, done = st
            mid = 0.5 * (lo + hi)
            cnt = count_bm(mid)
            feas = jnp.logical_and(cnt >= K, done == 0)
            best = jnp.where(feas, mid, best)
            lo2 = jnp.where(feas, mid, lo)
            hi2 = jnp.where(jnp.logical_or(feas, done != 0), hi, mid)
            done2 = jnp.where(jnp.logical_and(feas, cnt <= 160),
                              jnp.int32(1), done)
            return lo2, hi2, best, done2

        _, _, g_thr, _ = lax.fori_loop(
            0, 18, s1_body, (gmin, gmax, gmin, jnp.int32(0)))

        def append16(ref, pend, vals, off, cnt, trash, cap):
            sh = off & 15
            rot = [_perm16(v, (lane - sh) & 15) for v in vals]
            merged = [jnp.where(lane < sh, p, r) for p, r in zip(pend, rot)]
            full = (sh + cnt) >= 16
            dst = jnp.where(full, jnp.minimum(off - sh, cap), trash)
            for rf, m in zip(ref, merged):
                rf[pl.ds(dst, 16)] = m
            pend2 = [jnp.where(full, r, m) for r, m in zip(rot, merged)]
            return pend2, off + cnt

        def flush16(ref, pend, off, cap):
            sh = off & 15
            dst = jnp.minimum(off - sh, cap)
            for rf, p in zip(ref, pend):
                rf[pl.ds(dst, 16)] = p

        def qual_grp(grp, carry):
            qoff, pend = carry
            bmv = bm_v[pl.ds(grp * 16, 16)]
            msk = bmv >= g_thr
            (blks,), cnt = _compact16([grp * 16 + lane], msk, lane)
            (pend2,), qoff2 = append16([qblk_v], [pend], [blks], qoff, cnt,
                                       NBLK + 16, NBLK - 16)
            return qoff2, pend2

        nq, qpend = lax.fori_loop(0, NBLK // 16, qual_grp,
                                  (jnp.int32(0), lane))
        flush16([qblk_v], [qpend], nq, NBLK - 16)

        def coll_blk(q, carry):
            off, pk, pi = carry
            qv = qblk_v[pl.ds(q - (q & 15), 16)]
            blk = _allsum16(jnp.where(lane == (q & 15), qv, 0), lane)[0]
            for t in range(8):
                x = row_v[blk, pl.ds(t * 16, 16)]
                msk = x >= g_thr
                (sk, si), cnt = _compact16(
                    [_skey16(x), blk * 128 + t * 16 + lane], msk, lane)
                (pk, pi), off = append16([cmono_v, cidx_v], [pk, pi],
                                         [sk, si], off, cnt, CAP + 16,
                                         CAP - 16)
            return off, pk, pi

        n, cpk, cpi = plsc.parallel_loop(
            0, nq, carry=(jnp.int32(0), lane, lane))(coll_blk)
        flush16([cmono_v, cidx_v], [cpk, cpi], n, CAP - 16)
        nv = (n + 15) // 16

        def count_cand(t, strict):
            def cb(v, c):
                key = cmono_v[pl.ds(v * 16, 16)]
                valid = (v * 16 + lane) < n
                hit = jnp.where(strict, key > t, key >= t)
                return c + jnp.where(jnp.logical_and(valid, hit), 1, 0)
            cvec = plsc.parallel_loop(0, nv, carry=jnp.zeros((LN,), jnp.int32))(cb)
            return _allsum16(cvec, lane)[0]

        def s2_body(_, st):
            lo, hi = st
            mid = (lo & hi) + ((lo ^ hi) >> 1) + ((lo ^ hi) & 1)
            feas = jnp.logical_and(lo < hi, count_cand(mid, jnp.bool_(False)) >= K)
            return (jnp.where(feas, mid, lo),
                    jnp.where(jnp.logical_or(feas, lo >= hi), hi, mid - 1))

        tau, _ = lax.fori_loop(
            0, 32, s2_body,
            (jnp.int32(-2147483648), jnp.int32(2147483647)))
        c_gt = count_cand(tau, jnp.bool_(True))
        rem = K - c_gt

        def sel_body(v, carry):
            off2, eqseen, pk, pi = carry
            key = cmono_v[pl.ds(v * 16, 16)]
            valid = (v * 16 + lane) < n
            gt = jnp.logical_and(valid, key > tau)
            eq = jnp.logical_and(valid, key == tau)
            eqr = _cumsum16(jnp.where(eq, 1, 0), lane) + eqseen
            sel = jnp.logical_or(gt, jnp.logical_and(eq, eqr <= rem))
            idx16 = cidx_v[pl.ds(v * 16, 16)]
            (key_s, idx_s), cnt = _compact16([key, idx16], sel, lane)
            (pk, pi), off2b = append16([skey_v, sidx_v], [pk, pi],
                                       [key_s, idx_s], off2, cnt, 144, 96)
            return off2b, eqr[15], pk, pi

        off2, _, spk, spi = plsc.parallel_loop(
            0, nv, carry=(jnp.int32(1), jnp.int32(0), lane, lane))(sel_body)
        flush16([skey_v, sidx_v], [spk, spi], off2, 96)

        rlane = r & 15
        rbase = r - rlane
        tgt = _allsum16(jnp.where(lane == rlane, tgt_v[pl.ds(rbase, 16)], 0), lane)[0]
        tl = _allsum16(jnp.where(lane == rlane, tl_v[pl.ds(rbase, 16)], 0.0), lane)[0]
        first_idx = jnp.where(lane == 0, tgt, sidx_v[pl.ds(0, 16)])
        sidx_v[pl.ds(0, 16)] = first_idx
        for i in range(OUTW // 8 // 2):
            ix = sidx_v[pl.ds(i * 16, 16)]
            vv = _sval16(skey_v[pl.ds(i * 16, 16)])
            sval_v[pl.ds(i * 16, 16)] = jnp.where(ix == tgt, tl, vv)

        pltpu.sync_copy(sidx_v.at[pl.ds(0, 128)], idx_out.at[pl.ds(b * 128, 128)])
        pltpu.sync_copy(sval_v.at[pl.ds(0, 128)], val_out.at[pl.ds(b * 128, 128)])
        return 0

    lax.fori_loop(0, RPW, row_body, 0)


def _sc_topk(masked_logits, target_id, target_logit):
    fn = pl.kernel(
        _topk_body,
        out_type=[jax.ShapeDtypeStruct((B * 128,), jnp.int32),
                  jax.ShapeDtypeStruct((B * 128,), jnp.float32)],
        mesh=plsc.VectorSubcoreMesh(core_axis_name="c", subcore_axis_name="s"),
        scratch_types=[
            pltpu.VMEM((NBLK, 128), jnp.float32),
            pltpu.VMEM((NBLK,), jnp.float32),
            pltpu.VMEM((NBLK + 48,), jnp.int32),
            pltpu.VMEM((CAP + 48,), jnp.int32),
            pltpu.VMEM((CAP + 48,), jnp.int32),
            pltpu.VMEM((256,), jnp.int32),
            pltpu.VMEM((256,), jnp.int32),
            pltpu.VMEM((256,), jnp.float32),
            pltpu.VMEM((RPW,), jnp.int32),
            pltpu.VMEM((RPW,), jnp.float32),
            pltpu.SemaphoreType.DMA,
            pltpu.SemaphoreType.DMA,
        ],
    )
    return fn(masked_logits, target_id.astype(jnp.int32),
              target_logit.reshape(B))


def _loss_kernel(noise_ref, actual_ref, out_ref):
    noise = noise_ref[...]
    actual = actual_ref[...]

    def softmax(x):
        m = jnp.max(x, axis=1, keepdims=True)
        e = jnp.exp(x - m)
        return e / jnp.sum(e, axis=1, keepdims=True)

    n_sm = softmax(noise)
    a_sm = softmax(actual)
    deno = K * n_sm + a_sm
    tmp1 = a_sm / deno
    tmp2 = n_sm / deno
    pos = jax.lax.broadcasted_iota(jnp.int32, (B, K + 1), 1)
    likeli = jnp.where(pos == 0, tmp1, tmp2)
    loss = -jnp.sum(jnp.log(likeli)) / (B * (K + 1))
    out_ref[...] = loss.reshape(1, 1)


def _nce_loss(noise_raw, actual_raw):
    out = pl.pallas_call(
        _loss_kernel,
        out_shape=jax.ShapeDtypeStruct((1, 1), jnp.float32),
    )(noise_raw, actual_raw)
    return out[0, 0]


def _pool(E, item_seq, item_seq_len):
    h = jnp.take(E, item_seq, axis=0)
    mask = (jnp.arange(L)[None, :] < item_seq_len[:, None]).astype(h.dtype)
    denom = jnp.maximum(item_seq_len, 1).astype(h.dtype)[:, None]
    return (h * mask[:, :, None]).sum(axis=1) / denom


def kernel(Ep, Wp, Eq, Wq, item_seq, item_seq_len, target_id):
    pooled_q = _pool(Eq, item_seq, item_seq_len)
    pooled_p = _pool(Ep, item_seq, item_seq_len)

    masked_logits, target_logit = _masked_logits(pooled_q, Wq, target_id)

    idx_out, val_out = _sc_topk(masked_logits, target_id, target_logit)
    indices = idx_out.reshape(B, 128)[:, :K + 1]
    noise_raw = val_out.reshape(B, 128)[:, :K + 1]

    wp_rows = jnp.take(Wp, indices, axis=0)
    actual_raw = jnp.einsum("bd,bjd->bj", pooled_p, wp_rows)

    return _nce_loss(noise_raw, actual_raw)


# speedup vs baseline: 9.6492x; 1.0050x over previous
import functools

import numpy as np

import jax
import jax.numpy as jnp
from jax import lax
from jax.experimental import pallas as pl
from jax.experimental.pallas import tpu as pltpu
from jax.experimental.pallas import tpu_sc as plsc

B, L, V, D = 1024, 50, 100000, 64
K = 100
BT = 512
VT = 2048
GRID_V = (V + VT - 1) // VT


def _logits_kernel(pooled_ref, w_ref, tgt_ref, out_ref, tl_ref):
    j = pl.program_id(1)
    logits = jax.lax.dot_general(
        pooled_ref[...], w_ref[...],
        (((1,), (1,)), ((), ())),
        preferred_element_type=jnp.float32,
    )
    cols = j * VT + jax.lax.broadcasted_iota(jnp.int32, (BT, VT), 1)
    match = cols == tgt_ref[...]
    masked = jnp.where(match, 0.0, logits)
    masked = jnp.where(cols < V, masked, NEG)
    out_ref[...] = masked.reshape(BT, VT // 128, 128)
    contrib = jnp.sum(jnp.where(jnp.logical_and(match, cols < V), logits, 0.0),
                      axis=1, keepdims=True)

    @pl.when(j == 0)
    def _():
        tl_ref[...] = contrib

    @pl.when(j > 0)
    def _():
        tl_ref[...] += contrib


def _masked_logits(pooled_q, Wq, target_id):
    tgt2d = target_id.reshape(B, 1).astype(jnp.int32)
    return pl.pallas_call(
        _logits_kernel,
        grid=(B // BT, GRID_V),
        in_specs=[
            pl.BlockSpec((BT, D), lambda i, j: (i, 0)),
            pl.BlockSpec((VT, D), lambda i, j: (j, 0)),
            pl.BlockSpec((BT, 1), lambda i, j: (i, 0)),
        ],
        out_specs=[
            pl.BlockSpec((BT, VT // 128, 128), lambda i, j: (i, j, 0)),
            pl.BlockSpec((BT, 1), lambda i, j: (i, 0)),
        ],
        out_shape=[
            jax.ShapeDtypeStruct((B, VPAD // 128, 128), jnp.float32),
            jax.ShapeDtypeStruct((B, 1), jnp.float32),
        ],
    )(pooled_q, Wq, tgt2d)


NC, NS, LN = 2, 16, 16
NW = NC * NS
RPW = B // NW
NBLK = 784
VPAD = NBLK * 128
CAP = 2048
OUTW = 104
NEG = -3.0e38
TOPBIT = np.uint32(0x80000000)


_GATHER_DNUMS = lax.GatherDimensionNumbers(
    offset_dims=(), collapsed_slice_dims=(0,), start_index_map=(0,))


def _perm16(x, perm):
    return lax.gather(x, perm.reshape(16, 1), _GATHER_DNUMS, (1,),
                      mode=lax.GatherScatterMode.PROMISE_IN_BOUNDS)


def _allmax16(x, lane):
    for sh in (8, 4, 2, 1):
        x = jnp.maximum(x, _perm16(x, (lane + sh) & 15))
    return x


def _allmin16(x, lane):
    for sh in (8, 4, 2, 1):
        x = jnp.minimum(x, _perm16(x, (lane + sh) & 15))
    return x


def _allsum16(x, lane):
    for sh in (8, 4, 2, 1):
        x = x + _perm16(x, (lane + sh) & 15)
    return x


def _cumsum16(x, lane):
    for sh in (1, 2, 4, 8):
        shifted = _perm16(x, (lane - sh) & 15)
        x = x + jnp.where(lane >= sh, shifted, 0)
    return x


def _compact16(arrs, msk, lane):
    rank = _cumsum16(jnp.where(msk, 1, 0), lane)
    cnt = rank[15]
    d = jnp.where(msk, lane - (rank - 1), 0)
    for k in (1, 2, 4, 8):
        perm = (lane + k) & 15
        d_s = _perm16(d, perm)
        take = jnp.logical_and(lane < 16 - k, (d_s & k) != 0)
        arrs = [jnp.where(take, _perm16(a, perm), a) for a in arrs]
        d = jnp.where(take, d_s - k, jnp.where((d & k) != 0, 0, d))
    return arrs, cnt


def _skey16(x):
    u = lax.bitcast_convert_type(x, jnp.int32)
    return u ^ (lax.shift_right_arithmetic(u, 31) & jnp.int32(0x7FFFFFFF))


def _sval16(k):
    u = k ^ (lax.shift_right_arithmetic(k, 31) & jnp.int32(0x7FFFFFFF))
    return lax.bitcast_convert_type(u, jnp.float32)


def _topk_body(logits_hbm, tgt_hbm, tl_hbm, idx_out, val_out,
               row_v, bm_v, qblk_v, cmono_v, cidx_v, sidx_v, skey_v, sval_v,
               tgt_v, tl_v, sem):
    wid = lax.axis_index("s") * NC + lax.axis_index("c")
    lane = lax.iota(jnp.int32, 16)

    pltpu.sync_copy(tgt_hbm.at[pl.ds(wid * RPW, RPW)], tgt_v)
    pltpu.sync_copy(tl_hbm.at[pl.ds(wid * RPW, RPW)], tl_v)

    def row_body(r, _):
        b = wid * RPW + r
        pltpu.sync_copy(logits_hbm.at[b], row_v)

        def bm_group(g, carry):
            gminv, gmaxv = carry
            acc = jnp.full((LN,), NEG, jnp.float32)
            for j in range(16):
                m = row_v[g * 16 + j, pl.ds(0, 16)]
                for t in range(1, 8):
                    m = jnp.maximum(m, row_v[g * 16 + j, pl.ds(t * 16, 16)])
                s = _allmax16(m, lane)[0]
                acc = jnp.where(lane == j, s, acc)
            bm_v[pl.ds(g * 16, 16)] = acc
            valid = (g * 16 + lane) < (V + 127) // 128
            gminv = jnp.minimum(gminv, jnp.where(valid, acc, -NEG))
            gmaxv = jnp.maximum(gmaxv, jnp.where(valid, acc, NEG))
            return gminv, gmaxv

        gminv, gmaxv = lax.fori_loop(
            0, NBLK // 16, bm_group,
            (jnp.full((LN,), -NEG, jnp.float32), jnp.full((LN,), NEG, jnp.float32)))
        gmin = _allmin16(gminv, lane)[0]
        gmax = _allmax16(gmaxv, lane)[0]

        def count_bm(t):
            def cb(v, c):
                return c + jnp.where(bm_v[pl.ds(v * 16, 16)] >= t, 1, 0)
            return _allsum16(
                lax.fori_loop(0, NBLK // 16, cb, jnp.zeros((LN,), jnp.int32)), lane)[0]

        def s1_body(_, st):
            lo, hi, best, done = st
            mid = 0.5 * (lo + hi)
            cnt = count_bm(mid)
            feas = jnp.logical_and(cnt >= K, done == 0)
            best = jnp.where(feas, mid, best)
            lo2 = jnp.where(feas, mid, lo)
            hi2 = jnp.where(jnp.logical_or(feas, done != 0), hi, mid)
            done2 = jnp.where(jnp.logical_and(feas, cnt <= 160),
                              jnp.int32(1), done)
            return lo2, hi2, best, done2

        _, _, g_thr, _ = lax.fori_loop(
            0, 18, s1_body, (gmin, gmax, gmin, jnp.int32(0)))

        def append16(ref, pend, vals, off, cnt, trash, cap):
            sh = off & 15
            rot = [_perm16(v, (lane - sh) & 15) for v in vals]
            merged = [jnp.where(lane < sh, p, r) for p, r in zip(pend, rot)]
            full = (sh + cnt) >= 16
            dst = jnp.where(full, jnp.minimum(off - sh, cap), trash)
            for rf, m in zip(ref, merged):
                rf[pl.ds(dst, 16)] = m
            pend2 = [jnp.where(full, r, m) for r, m in zip(rot, merged)]
            return pend2, off + cnt

        def flush16(ref, pend, off, cap):
            sh = off & 15
            dst = jnp.minimum(off - sh, cap)
            for rf, p in zip(ref, pend):
                rf[pl.ds(dst, 16)] = p

        def qual_grp(grp, carry):
            qoff, pend = carry
            bmv = bm_v[pl.ds(grp * 16, 16)]
            msk = bmv >= g_thr
            (blks,), cnt = _compact16([grp * 16 + lane], msk, lane)
            (pend2,), qoff2 = append16([qblk_v], [pend], [blks], qoff, cnt,
                                       NBLK + 16, NBLK - 16)
            return qoff2, pend2

        nq, qpend = lax.fori_loop(0, NBLK // 16, qual_grp,
                                  (jnp.int32(0), lane))
        flush16([qblk_v], [qpend], nq, NBLK - 16)

        def coll_blk(q, carry):
            off, pk, pi = carry
            qv = qblk_v[pl.ds(q - (q & 15), 16)]
            blk = _allsum16(jnp.where(lane == (q & 15), qv, 0), lane)[0]
            for t in range(8):
                x = row_v[blk, pl.ds(t * 16, 16)]
                msk = x >= g_thr
                (sk, si), cnt = _compact16(
                    [_skey16(x), blk * 128 + t * 16 + lane], msk, lane)
                (pk, pi), off = append16([cmono_v, cidx_v], [pk, pi],
                                         [sk, si], off, cnt, CAP + 16,
                                         CAP - 16)
            return off, pk, pi

        n, cpk, cpi = plsc.parallel_loop(
            0, nq, carry=(jnp.int32(0), lane, lane))(coll_blk)
        flush16([cmono_v, cidx_v], [cpk, cpi], n, CAP - 16)
        nv = (n + 15) // 16

        def count_cand(t, strict):
            def cb(v, c):
                key = cmono_v[pl.ds(v * 16, 16)]
                valid = (v * 16 + lane) < n
                hit = jnp.where(strict, key > t, key >= t)
                return c + jnp.where(jnp.logical_and(valid, hit), 1, 0)
            cvec = plsc.parallel_loop(0, nv, carry=jnp.zeros((LN,), jnp.int32))(cb)
            return _allsum16(cvec, lane)[0]

        def s2_body(_, st):
            lo, hi = st
            mid = (lo & hi) + ((lo ^ hi) >> 1) + ((lo ^ hi) & 1)
            feas = jnp.logical_and(lo < hi, count_cand(mid, jnp.bool_(False)) >= K)
            return (jnp.where(feas, mid, lo),
                    jnp.where(jnp.logical_or(feas, lo >= hi), hi, mid - 1))

        tau, _ = lax.fori_loop(
            0, 32, s2_body,
            (jnp.int32(-2147483648), jnp.int32(2147483647)))
        c_gt = count_cand(tau, jnp.bool_(True))
        rem = K - c_gt

        def sel_body(v, carry):
            off2, eqseen, pk, pi = carry
            key = cmono_v[pl.ds(v * 16, 16)]
            valid = (v * 16 + lane) < n
            gt = jnp.logical_and(valid, key > tau)
            eq = jnp.logical_and(valid, key == tau)
            eqr = _cumsum16(jnp.where(eq, 1, 0), lane) + eqseen
            sel = jnp.logical_or(gt, jnp.logical_and(eq, eqr <= rem))
            idx16 = cidx_v[pl.ds(v * 16, 16)]
            (key_s, idx_s), cnt = _compact16([key, idx16], sel, lane)
            (pk, pi), off2b = append16([skey_v, sidx_v], [pk, pi],
                                       [key_s, idx_s], off2, cnt, 144, 96)
            return off2b, eqr[15], pk, pi

        off2, _, spk, spi = plsc.parallel_loop(
            0, nv, carry=(jnp.int32(1), jnp.int32(0), lane, lane))(sel_body)
        flush16([skey_v, sidx_v], [spk, spi], off2, 96)

        rlane = r & 15
        rbase = r - rlane
        tgt = _allsum16(jnp.where(lane == rlane, tgt_v[pl.ds(rbase, 16)], 0), lane)[0]
        tl = _allsum16(jnp.where(lane == rlane, tl_v[pl.ds(rbase, 16)], 0.0), lane)[0]
        first_idx = jnp.where(lane == 0, tgt, sidx_v[pl.ds(0, 16)])
        sidx_v[pl.ds(0, 16)] = first_idx
        for i in range(OUTW // 8 // 2):
            ix = sidx_v[pl.ds(i * 16, 16)]
            vv = _sval16(skey_v[pl.ds(i * 16, 16)])
            sval_v[pl.ds(i * 16, 16)] = jnp.where(ix == tgt, tl, vv)

        pltpu.sync_copy(sidx_v.at[pl.ds(0, 128)], idx_out.at[pl.ds(b * 128, 128)])
        pltpu.sync_copy(sval_v.at[pl.ds(0, 128)], val_out.at[pl.ds(b * 128, 128)])
        return 0

    lax.fori_loop(0, RPW, row_body, 0)


def _sc_topk(masked_logits, target_id, target_logit):
    fn = pl.kernel(
        _topk_body,
        out_type=[jax.ShapeDtypeStruct((B * 128,), jnp.int32),
                  jax.ShapeDtypeStruct((B * 128,), jnp.float32)],
        mesh=plsc.VectorSubcoreMesh(core_axis_name="c", subcore_axis_name="s"),
        scratch_types=[
            pltpu.VMEM((NBLK, 128), jnp.float32),
            pltpu.VMEM((NBLK,), jnp.float32),
            pltpu.VMEM((NBLK + 48,), jnp.int32),
            pltpu.VMEM((CAP + 48,), jnp.int32),
            pltpu.VMEM((CAP + 48,), jnp.int32),
            pltpu.VMEM((256,), jnp.int32),
            pltpu.VMEM((256,), jnp.int32),
            pltpu.VMEM((256,), jnp.float32),
            pltpu.VMEM((RPW,), jnp.int32),
            pltpu.VMEM((RPW,), jnp.float32),
            pltpu.SemaphoreType.DMA,
        ],
    )
    return fn(masked_logits, target_id.astype(jnp.int32),
              target_logit.reshape(B))


def _loss_kernel(noise_ref, actual_ref, out_ref):
    noise = noise_ref[...]
    actual = actual_ref[...]

    def softmax(x):
        m = jnp.max(x, axis=1, keepdims=True)
        e = jnp.exp(x - m)
        return e / jnp.sum(e, axis=1, keepdims=True)

    n_sm = softmax(noise)
    a_sm = softmax(actual)
    deno = K * n_sm + a_sm
    tmp1 = a_sm / deno
    tmp2 = n_sm / deno
    pos = jax.lax.broadcasted_iota(jnp.int32, (B, K + 1), 1)
    likeli = jnp.where(pos == 0, tmp1, tmp2)
    loss = -jnp.sum(jnp.log(likeli)) / (B * (K + 1))
    out_ref[...] = loss.reshape(1, 1)


def _nce_loss(noise_raw, actual_raw):
    out = pl.pallas_call(
        _loss_kernel,
        out_shape=jax.ShapeDtypeStruct((1, 1), jnp.float32),
    )(noise_raw, actual_raw)
    return out[0, 0]


def _pool(E, item_seq, item_seq_len):
    h = jnp.take(E, item_seq, axis=0)
    mask = (jnp.arange(L)[None, :] < item_seq_len[:, None]).astype(h.dtype)
    denom = jnp.maximum(item_seq_len, 1).astype(h.dtype)[:, None]
    return (h * mask[:, :, None]).sum(axis=1) / denom


def kernel(Ep, Wp, Eq, Wq, item_seq, item_seq_len, target_id):
    pooled_q = _pool(Eq, item_seq, item_seq_len)
    pooled_p = _pool(Ep, item_seq, item_seq_len)

    masked_logits, target_logit = _masked_logits(pooled_q, Wq, target_id)

    idx_out, val_out = _sc_topk(masked_logits, target_id, target_logit)
    indices = idx_out.reshape(B, 128)[:, :K + 1]
    noise_raw = val_out.reshape(B, 128)[:, :K + 1]

    wp_rows = jnp.take(Wp, indices, axis=0)
    actual_raw = jnp.einsum("bd,bjd->bj", pooled_p, wp_rows)

    return _nce_loss(noise_raw, actual_raw)
